# speedup vs baseline: 6.1084x; 6.1084x over previous
import functools
import math

import jax
import jax.numpy as jnp
from jax import lax
from jax.experimental import pallas as pl
from jax.experimental.pallas import tpu as pltpu
from jax.experimental.pallas import tpu_sc as plsc

D = 128
SCALE = math.sqrt(float(D))


def _scale_body(t_ref, o_ref):
    o_ref[...] = t_ref[...] * SCALE


@jax.jit
def _scale_table(table):
    v = table.shape[0]
    blk = 2048
    grid = pl.cdiv(v, blk)
    return pl.pallas_call(
        _scale_body,
        grid=(grid,),
        in_specs=[pl.BlockSpec((blk, D), lambda i: (i, 0))],
        out_specs=pl.BlockSpec((blk, D), lambda i: (i, 0)),
        out_shape=jax.ShapeDtypeStruct((v, D), jnp.float32),
    )(table)


@functools.lru_cache()
def _make_gather(b_total):
    info = plsc.get_sparse_core_info()
    nc, ns = info.num_cores, info.num_subcores
    nw = nc * ns
    assert b_total % nw == 0
    b_per_w = b_total // nw
    chunk = 256
    assert b_per_w % chunk == 0
    n_chunks = b_per_w // chunk
    mesh = plsc.VectorSubcoreMesh(core_axis_name="c", subcore_axis_name="s")

    @functools.partial(
        pl.kernel,
        mesh=mesh,
        out_type=jax.ShapeDtypeStruct((b_total, D), jnp.float32),
        scratch_types=[
            pltpu.VMEM((chunk,), jnp.int32),
            pltpu.VMEM((chunk, D), jnp.float32),
            pltpu.SemaphoreType.DMA,
        ],
    )
    def gather_kernel(table_hbm, idx_hbm, out_hbm, idx_v, rows_v, sem):
        wid = lax.axis_index("s") * nc + lax.axis_index("c")
        base = wid * b_per_w

        def body(i, carry):
            off = base + i * chunk
            pltpu.sync_copy(idx_hbm.at[pl.ds(off, chunk)], idx_v)
            pltpu.async_copy(table_hbm.at[idx_v], rows_v, sem).wait()
            pltpu.sync_copy(rows_v, out_hbm.at[pl.ds(off, chunk)])
            return carry

        lax.fori_loop(0, n_chunks, body, 0)

    return gather_kernel


def kernel(x, table):
    b0, b1 = x.shape
    idx = x.reshape(b0 * b1).astype(jnp.int32)
    scaled = _scale_table(table)
    out = _make_gather(b0 * b1)(scaled, idx)
    return out.reshape(b0, b1, D)


# speedup vs baseline: 7.9519x; 1.3018x over previous
import functools
import math

import jax
import jax.numpy as jnp
from jax import lax
from jax.experimental import pallas as pl
from jax.experimental.pallas import tpu as pltpu
from jax.experimental.pallas import tpu_sc as plsc

D = 128
SCALE = math.sqrt(float(D))


def _scale_body(t_ref, o_ref):
    o_ref[...] = t_ref[...] * SCALE


@jax.jit
def _scale_table(table):
    v = table.shape[0]
    blk = 2048
    grid = pl.cdiv(v, blk)
    return pl.pallas_call(
        _scale_body,
        grid=(grid,),
        in_specs=[pl.BlockSpec((blk, D), lambda i: (i, 0))],
        out_specs=pl.BlockSpec((blk, D), lambda i: (i, 0)),
        out_shape=jax.ShapeDtypeStruct((v, D), jnp.float32),
    )(table)


@functools.lru_cache()
def _make_gather(b_total):
    info = plsc.get_sparse_core_info()
    nc, ns = info.num_cores, info.num_subcores
    nw = nc * ns
    assert b_total % nw == 0
    b_per_w = b_total // nw
    chunk = 256
    assert b_per_w % (2 * chunk) == 0
    n_pairs = b_per_w // (2 * chunk)
    mesh = plsc.VectorSubcoreMesh(core_axis_name="c", subcore_axis_name="s")

    @functools.partial(
        pl.kernel,
        mesh=mesh,
        out_type=jax.ShapeDtypeStruct((b_total, D), jnp.float32),
        scratch_types=[
            pltpu.VMEM((b_per_w,), jnp.int32),
            pltpu.VMEM((chunk, D), jnp.float32),
            pltpu.VMEM((chunk, D), jnp.float32),
            pltpu.SemaphoreType.DMA,
            pltpu.SemaphoreType.DMA,
            pltpu.SemaphoreType.DMA,
            pltpu.SemaphoreType.DMA,
        ],
    )
    def gather_kernel(table_hbm, idx_hbm, out_hbm, idx_v, rows0, rows1,
                      g0, g1, s0, s1):
        wid = lax.axis_index("s") * nc + lax.axis_index("c")
        base = wid * b_per_w
        pltpu.sync_copy(idx_hbm.at[pl.ds(base, b_per_w)], idx_v)

        def gstart(i, buf, sem):
            pltpu.async_copy(
                table_hbm.at[idx_v.at[pl.ds(i * chunk, chunk)]], buf, sem)

        def gwait(i, buf, sem):
            pltpu.make_async_copy(
                table_hbm.at[idx_v.at[pl.ds(i * chunk, chunk)]], buf,
                sem).wait()

        def sstart(i, buf, sem):
            pltpu.async_copy(
                buf, out_hbm.at[pl.ds(base + i * chunk, chunk)], sem)

        def swait(i, buf, sem):
            pltpu.make_async_copy(
                buf, out_hbm.at[pl.ds(base + i * chunk, chunk)], sem).wait()

        gstart(0, rows0, g0)

        def body(j, carry):
            i0 = 2 * j
            i1 = i0 + 1

            @pl.when(j > 0)
            def _():
                swait(i0 - 1, rows1, s1)

            gstart(i1, rows1, g1)
            gwait(i0, rows0, g0)
            sstart(i0, rows0, s0)

            @pl.when(j < n_pairs - 1)
            def _():
                swait(i0, rows0, s0)
                gstart(i0 + 2, rows0, g0)

            gwait(i1, rows1, g1)
            sstart(i1, rows1, s1)
            return carry

        lax.fori_loop(0, n_pairs, body, 0)
        swait(2 * n_pairs - 2, rows0, s0)
        swait(2 * n_pairs - 1, rows1, s1)

    return gather_kernel


def kernel(x, table):
    b0, b1 = x.shape
    idx = x.reshape(b0 * b1).astype(jnp.int32)
    scaled = _scale_table(table)
    out = _make_gather(b0 * b1)(scaled, idx)
    return out.reshape(b0, b1, D)


# speedup vs baseline: 7.9531x; 1.0002x over previous
import functools
import math

import jax
import jax.numpy as jnp
from jax import lax
from jax.experimental import pallas as pl
from jax.experimental.pallas import tpu as pltpu
from jax.experimental.pallas import tpu_sc as plsc

D = 128
SCALE = math.sqrt(float(D))


def _scale_body(t_ref, o_ref):
    o_ref[...] = t_ref[...] * SCALE


@jax.jit
def _scale_table(table):
    v = table.shape[0]
    blk = 2048
    grid = pl.cdiv(v, blk)
    return pl.pallas_call(
        _scale_body,
        grid=(grid,),
        in_specs=[pl.BlockSpec((blk, D), lambda i: (i, 0))],
        out_specs=pl.BlockSpec((blk, D), lambda i: (i, 0)),
        out_shape=jax.ShapeDtypeStruct((v, D), jnp.float32),
    )(table)


@functools.lru_cache()
def _make_gather(b_total):
    info = plsc.get_sparse_core_info()
    nc, ns = info.num_cores, info.num_subcores
    nw = nc * ns
    assert b_total % nw == 0
    b_per_w = b_total // nw
    chunk = 200
    nbuf = 4
    assert b_per_w % (nbuf * chunk) == 0
    n_chunks = b_per_w // chunk
    n_groups = n_chunks // nbuf
    mesh = plsc.VectorSubcoreMesh(core_axis_name="c", subcore_axis_name="s")

    @functools.partial(
        pl.kernel,
        mesh=mesh,
        out_type=jax.ShapeDtypeStruct((b_total, D), jnp.float32),
        scratch_types=[
            pltpu.VMEM((b_per_w,), jnp.int32),
        ] + [pltpu.VMEM((chunk, D), jnp.float32)] * nbuf
          + [pltpu.SemaphoreType.DMA] * (2 * nbuf),
    )
    def gather_kernel(table_hbm, idx_hbm, out_hbm, idx_v, *rest):
        rows = rest[:nbuf]
        gsem = rest[nbuf:2 * nbuf]
        ssem = rest[2 * nbuf:]
        wid = lax.axis_index("s") * nc + lax.axis_index("c")
        base = wid * b_per_w
        pltpu.sync_copy(idx_hbm.at[pl.ds(base, b_per_w)], idx_v)

        def gstart(i, b):
            pltpu.async_copy(
                table_hbm.at[idx_v.at[pl.ds(i * chunk, chunk)]], rows[b],
                gsem[b])

        def gwait(i, b):
            pltpu.make_async_copy(
                table_hbm.at[idx_v.at[pl.ds(i * chunk, chunk)]], rows[b],
                gsem[b]).wait()

        def sstart(i, b):
            pltpu.async_copy(
                rows[b], out_hbm.at[pl.ds(base + i * chunk, chunk)], ssem[b])

        def swait(i, b):
            pltpu.make_async_copy(
                rows[b], out_hbm.at[pl.ds(base + i * chunk, chunk)],
                ssem[b]).wait()

        for b in range(nbuf - 1):
            gstart(b, b)

        def body(g, carry):
            for b in range(nbuf):
                i = g * nbuf + b
                pb = (b - 1) % nbuf
                gwait(i, b)
                sstart(i, b)

                @pl.when(i > 0)
                def _():
                    swait(i - 1, pb)

                @pl.when(i + nbuf - 1 < n_chunks)
                def _():
                    gstart(i + nbuf - 1, pb)
            return carry

        lax.fori_loop(0, n_groups, body, 0)
        swait(n_chunks - 1, nbuf - 1)

    return gather_kernel


def kernel(x, table):
    b0, b1 = x.shape
    idx = x.reshape(b0 * b1).astype(jnp.int32)
    scaled = _scale_table(table)
    out = _make_gather(b0 * b1)(scaled, idx)
    return out.reshape(b0, b1, D)


# speedup vs baseline: 9.1454x; 1.1499x over previous
import functools
import math

import jax
import jax.numpy as jnp
from jax import lax
from jax.experimental import pallas as pl
from jax.experimental.pallas import tpu as pltpu
from jax.experimental.pallas import tpu_sc as plsc

D = 128
SCALE = math.sqrt(float(D))


@functools.lru_cache()
def _make_gather(b_total):
    info = plsc.get_sparse_core_info()
    nc, ns = info.num_cores, info.num_subcores
    nw = nc * ns
    assert b_total % nw == 0
    b_per_w = b_total // nw
    chunk = 200
    nbuf = 4
    assert b_per_w % (nbuf * chunk) == 0
    n_chunks = b_per_w // chunk
    n_groups = n_chunks // nbuf
    mesh = plsc.VectorSubcoreMesh(core_axis_name="c", subcore_axis_name="s")

    @functools.partial(
        pl.kernel,
        mesh=mesh,
        out_type=jax.ShapeDtypeStruct((b_total, D), jnp.float32),
        scratch_types=[
            pltpu.VMEM((b_per_w,), jnp.int32),
        ] + [pltpu.VMEM((chunk, D), jnp.float32)] * nbuf
          + [pltpu.SemaphoreType.DMA] * (2 * nbuf),
    )
    def gather_kernel(table_hbm, idx_hbm, out_hbm, idx_v, *rest):
        rows = rest[:nbuf]
        gsem = rest[nbuf:2 * nbuf]
        ssem = rest[2 * nbuf:]
        wid = lax.axis_index("s") * nc + lax.axis_index("c")
        base = wid * b_per_w
        pltpu.sync_copy(idx_hbm.at[pl.ds(base, b_per_w)], idx_v)

        def gstart(i, b):
            pltpu.async_copy(
                table_hbm.at[idx_v.at[pl.ds(i * chunk, chunk)]], rows[b],
                gsem[b])

        def gwait(i, b):
            pltpu.make_async_copy(
                table_hbm.at[idx_v.at[pl.ds(i * chunk, chunk)]], rows[b],
                gsem[b]).wait()

        def sstart(i, b):
            pltpu.async_copy(
                rows[b], out_hbm.at[pl.ds(base + i * chunk, chunk)], ssem[b])

        def swait(i, b):
            pltpu.make_async_copy(
                rows[b], out_hbm.at[pl.ds(base + i * chunk, chunk)],
                ssem[b]).wait()

        for b in range(nbuf - 1):
            gstart(b, b)

        def body(g, carry):
            for b in range(nbuf):
                i = g * nbuf + b
                pb = (b - 1) % nbuf
                gwait(i, b)
                buf = rows[b]

                @plsc.parallel_loop(0, chunk, unroll=2)
                def _(r):
                    for c in range(D // 16):
                        sl = pl.ds(c * 16, 16)
                        buf[r, sl] = buf[r, sl] * SCALE

                sstart(i, b)

                @pl.when(i > 0)
                def _():
                    swait(i - 1, pb)

                @pl.when(i + nbuf - 1 < n_chunks)
                def _():
                    gstart(i + nbuf - 1, pb)
            return carry

        lax.fori_loop(0, n_groups, body, 0)
        swait(n_chunks - 1, nbuf - 1)

    return gather_kernel


def kernel(x, table):
    b0, b1 = x.shape
    idx = x.reshape(b0 * b1).astype(jnp.int32)
    out = _make_gather(b0 * b1)(table, idx)

    return out.reshape(b0, b1, D)


# speedup vs baseline: 9.1958x; 1.0055x over previous
import functools
import math

import jax
import jax.numpy as jnp
from jax import lax
from jax.experimental import pallas as pl
from jax.experimental.pallas import tpu as pltpu
from jax.experimental.pallas import tpu_sc as plsc

D = 128
SCALE = math.sqrt(float(D))


@functools.lru_cache()
def _make_gather(b_total):
    info = plsc.get_sparse_core_info()
    nc, ns = info.num_cores, info.num_subcores
    nw = nc * ns
    assert b_total % nw == 0
    b_per_w = b_total // nw
    chunk = 160
    nbuf = 5
    assert b_per_w % (nbuf * chunk) == 0
    n_chunks = b_per_w // chunk
    n_groups = n_chunks // nbuf
    mesh = plsc.VectorSubcoreMesh(core_axis_name="c", subcore_axis_name="s")

    @functools.partial(
        pl.kernel,
        mesh=mesh,
        out_type=jax.ShapeDtypeStruct((b_total, D), jnp.float32),
        scratch_types=[
            pltpu.VMEM((b_per_w,), jnp.int32),
        ] + [pltpu.VMEM((chunk, D), jnp.float32)] * nbuf
          + [pltpu.SemaphoreType.DMA] * (2 * nbuf),
    )
    def gather_kernel(table_hbm, idx_hbm, out_hbm, idx_v, *rest):
        rows = rest[:nbuf]
        gsem = rest[nbuf:2 * nbuf]
        ssem = rest[2 * nbuf:]
        wid = lax.axis_index("s") * nc + lax.axis_index("c")
        base = wid * b_per_w
        pltpu.sync_copy(idx_hbm.at[pl.ds(base, b_per_w)], idx_v)

        def gstart(i, b):
            pltpu.async_copy(
                table_hbm.at[idx_v.at[pl.ds(i * chunk, chunk)]], rows[b],
                gsem[b])

        def gwait(i, b):
            pltpu.make_async_copy(
                table_hbm.at[idx_v.at[pl.ds(i * chunk, chunk)]], rows[b],
                gsem[b]).wait()

        def sstart(i, b):
            pltpu.async_copy(
                rows[b], out_hbm.at[pl.ds(base + i * chunk, chunk)], ssem[b])

        def swait(i, b):
            pltpu.make_async_copy(
                rows[b], out_hbm.at[pl.ds(base + i * chunk, chunk)],
                ssem[b]).wait()

        for b in range(nbuf - 1):
            gstart(b, b)

        def body(g, carry):
            for b in range(nbuf):
                i = g * nbuf + b
                pb = (b - 1) % nbuf
                gwait(i, b)
                buf = rows[b]

                @plsc.parallel_loop(0, chunk, unroll=2)
                def _(r):
                    for c in range(D // 16):
                        sl = pl.ds(c * 16, 16)
                        buf[r, sl] = buf[r, sl] * SCALE

                sstart(i, b)

                @pl.when(i > 0)
                def _():
                    swait(i - 1, pb)

                @pl.when(i + nbuf - 1 < n_chunks)
                def _():
                    gstart(i + nbuf - 1, pb)
            return carry

        lax.fori_loop(0, n_groups, body, 0)
        swait(n_chunks - 1, nbuf - 1)

    return gather_kernel


def kernel(x, table):
    b0, b1 = x.shape
    idx = x.reshape(b0 * b1).astype(jnp.int32)
    out = _make_gather(b0 * b1)(table, idx)

    return out.reshape(b0, b1, D)
